# baseline (speedup 1.0000x reference)
import functools

import jax
import jax.numpy as jnp
from jax import lax
from jax.experimental import pallas as pl
from jax.experimental.pallas import tpu as pltpu
from jax.experimental.pallas import tpu_sc as plsc

N = 10000
E = 320000
H = 128
A = 100
B_E = 8000

_NTILE = 16
_EPT = E // _NTILE
_K = 40
_NCHUNK = _EPT // _K
_NB2 = _NCHUNK // 2
_KPQ = 80
_NHALF = N // 2
_DEN_OFF = 5120
_TRASH = 10120
_ACC = 10240
_TROWS = _ACC // _NTILE
_EPT_PQ = E // 32


def _embed_h_body(x_ref, w_ref, b_ref, o_ref):
    o_ref[...] = jnp.dot(x_ref[...], w_ref[...], preferred_element_type=jnp.float32) + b_ref[...]


def _node_mm_body(x_ref, w_ref, b_ref, ah_ref, bh_ref, dh_ref, eh_ref):
    y = jnp.dot(x_ref[...], w_ref[...], preferred_element_type=jnp.float32) + b_ref[...]
    ah_ref[...] = y[:, 0:H]
    bh_ref[...] = y[:, H:2 * H]
    dh_ref[...] = y[:, 2 * H:3 * H]
    eh_ref[...] = y[:, 3 * H:4 * H]


def _node_update_body(ah_ref, num_ref, den_ref, hres_ref, o_ref):
    hn = ah_ref[...] + num_ref[...] / (den_ref[...] + 1e-6)
    mu = jnp.mean(hn, axis=0, keepdims=True)
    var = jnp.mean((hn - mu) ** 2, axis=0, keepdims=True)
    o_ref[...] = hres_ref[...] + jax.nn.relu((hn - mu) * lax.rsqrt(var + 1e-5))


def _edge_embed_body(e_ref, we_ref, be_ref, w2_ref, b2_ref, e0_ref, ce_ref):
    e0 = jnp.dot(e_ref[...], we_ref[...], preferred_element_type=jnp.float32) + be_ref[...]
    e0_ref[...] = e0
    ce_ref[...] = jnp.dot(e0, w2_ref[...], preferred_element_type=jnp.float32) + b2_ref[...]


def _edge_update_body(eres_ref, ep_ref, mu_ref, rstd_ref, w2_ref, b2_ref,
                      eo_ref, ce_ref):
    eo = eres_ref[...] + jax.nn.relu((ep_ref[...] - mu_ref[...]) * rstd_ref[...])
    eo_ref[...] = eo
    ce_ref[...] = jnp.dot(eo, w2_ref[...], preferred_element_type=jnp.float32) + b2_ref[...]


def _bilin_s_body(x_ref, wa_ref, ba_ref, s_ref):
    z = jnp.dot(x_ref[...], wa_ref[...], preferred_element_type=jnp.float32) + ba_ref[...]
    m = jnp.max(z, axis=-1, keepdims=True)
    ez = jnp.exp(z - m)
    s_ref[...] = ez / jnp.sum(ez, axis=-1, keepdims=True)


def _bilin_update_body(h_ref, s_ref, st_ref, o_ref):
    m = jnp.dot(st_ref[...], h_ref[...], preferred_element_type=jnp.float32)
    o_ref[...] = h_ref[...] + jnp.dot(s_ref[...], m, preferred_element_type=jnp.float32)


def _readout_pq_body(h_ref, w0_ref, p_ref, q_ref):
    p_ref[...] = jnp.dot(h_ref[...], w0_ref[0:H], preferred_element_type=jnp.float32)
    q_ref[...] = jnp.dot(h_ref[...], w0_ref[H:2 * H], preferred_element_type=jnp.float32)


def _mlp_body(xp_ref, b0_ref, w1_ref, b1_ref, w2_ref, b2_ref, o_ref):
    x = jax.nn.relu(xp_ref[...] + b0_ref[...])
    y = jax.nn.relu(jnp.dot(x, w1_ref[...], preferred_element_type=jnp.float32) + b1_ref[...])
    o_ref[...] = jnp.dot(y, w2_ref[...], preferred_element_type=jnp.float32) + b2_ref[...]


def _embed_h(h, W, b):
    return pl.pallas_call(
        _embed_h_body,
        out_shape=jax.ShapeDtypeStruct((N, H), jnp.float32),
    )(h, W, b.reshape(1, H))


def _node_mm(x, Wstk, bstk):
    return pl.pallas_call(
        _node_mm_body,
        out_shape=[jax.ShapeDtypeStruct((N, H), jnp.float32)] * 4,
    )(x, Wstk, bstk)


def _node_update(Ah, num, den, hres):
    return pl.pallas_call(
        _node_update_body,
        out_shape=jax.ShapeDtypeStruct((N, H), jnp.float32),
    )(Ah, num, den, hres)


def _edge_embed(e, Wemb, bemb, W2, b2):
    g = E // B_E
    return pl.pallas_call(
        _edge_embed_body,
        grid=(g,),
        in_specs=[
            pl.BlockSpec((B_E, 16), lambda i: (i, 0)),
            pl.BlockSpec((16, H), lambda i: (0, 0)),
            pl.BlockSpec((1, H), lambda i: (0, 0)),
            pl.BlockSpec((H, H), lambda i: (0, 0)),
            pl.BlockSpec((1, H), lambda i: (0, 0)),
        ],
        out_specs=[
            pl.BlockSpec((B_E, H), lambda i: (i, 0)),
            pl.BlockSpec((B_E, H), lambda i: (i, 0)),
        ],
        out_shape=[
            jax.ShapeDtypeStruct((E, H), jnp.float32),
            jax.ShapeDtypeStruct((E, H), jnp.float32),
        ],
    )(e, Wemb, bemb.reshape(1, H), W2, b2.reshape(1, H))


def _edge_update(eres, ep, mu, rstd, W2n, b2n):
    g = E // B_E
    return pl.pallas_call(
        _edge_update_body,
        grid=(g,),
        in_specs=[
            pl.BlockSpec((B_E, H), lambda i: (i, 0)),
            pl.BlockSpec((B_E, H), lambda i: (i, 0)),
            pl.BlockSpec((1, H), lambda i: (0, 0)),
            pl.BlockSpec((1, H), lambda i: (0, 0)),
            pl.BlockSpec((H, H), lambda i: (0, 0)),
            pl.BlockSpec((1, H), lambda i: (0, 0)),
        ],
        out_specs=[
            pl.BlockSpec((B_E, H), lambda i: (i, 0)),
            pl.BlockSpec((B_E, H), lambda i: (i, 0)),
        ],
        out_shape=[
            jax.ShapeDtypeStruct((E, H), jnp.float32),
            jax.ShapeDtypeStruct((E, H), jnp.float32),
        ],
    )(eres, ep, mu, rstd, W2n, b2n.reshape(1, H))


def _bilinear(h, Wa, ba):
    s = pl.pallas_call(
        _bilin_s_body,
        out_shape=jax.ShapeDtypeStruct((N, A), jnp.float32),
    )(h, Wa, ba.reshape(1, A))
    h_out = pl.pallas_call(
        _bilin_update_body,
        out_shape=jax.ShapeDtypeStruct((N, H), jnp.float32),
    )(h, s, s.T)
    return h_out, s


def _readout_pq(h, W0):
    return pl.pallas_call(
        _readout_pq_body,
        out_shape=[jax.ShapeDtypeStruct((N, H), jnp.float32)] * 2,
    )(h, W0)


def _readout_mlp(xp, b0, W1, b1, W2, b2):
    g = E // B_E
    return pl.pallas_call(
        _mlp_body,
        grid=(g,),
        in_specs=[
            pl.BlockSpec((B_E, H), lambda i: (i, 0)),
            pl.BlockSpec((1, H), lambda i: (0, 0)),
            pl.BlockSpec((H, H // 2), lambda i: (0, 0)),
            pl.BlockSpec((1, H // 2), lambda i: (0, 0)),
            pl.BlockSpec((H // 2, 2), lambda i: (0, 0)),
            pl.BlockSpec((1, 2), lambda i: (0, 0)),
        ],
        out_specs=pl.BlockSpec((B_E, 2), lambda i: (i, 0)),
        out_shape=jax.ShapeDtypeStruct((E, 2), jnp.float32),
    )(xp, b0.reshape(1, H), W1, b1.reshape(1, H // 2), W2, b2.reshape(1, 2))


def _sc_edge_body(want_epre, bh_hbm, dh_hbm, eh_hbm, ce_hbm, src_hbm, dst_hbm,
                  *refs):
    if want_epre:
        nd_hbm, ep_hbm, st_hbm = refs[:3]
        r = refs[3:]
    else:
        nd_hbm = refs[0]
        ep_hbm = st_hbm = None
        r = refs[1:]
    slots = (r[0:8], r[8:16])
    statv, acc = r[16], r[17]
    si = (r[18], r[19])
    sg = (r[20], r[21])
    sw = (r[22], r[23])
    sep = (r[24], r[25])
    cid = lax.axis_index("c")
    sid = lax.axis_index("s")

    dhv0 = slots[0][4]

    def zrow(i, _):
        for j in range(8):
            dhv0[i, pl.ds(j * 16, 16)] = jnp.zeros((16,), jnp.float32)
        return 0
    lax.fori_loop(0, _K, zrow, 0)
    for rr in range(_TROWS // _K):
        pltpu.sync_copy(dhv0, acc.at[pl.ds(sid * _TROWS + rr * _K, _K)])
    plsc.subcore_barrier()

    base0 = sid * _EPT
    lo = cid * _NHALF

    def fire_idx(eb, s):
        srcv, dstv = slots[s][0], slots[s][1]
        pltpu.async_copy(src_hbm.at[pl.ds(eb, _K)], srcv, si[s])
        pltpu.async_copy(dst_hbm.at[pl.ds(eb, _K)], dstv, si[s])

    def wait_idx(s):
        srcv, dstv = slots[s][0], slots[s][1]
        pltpu.make_async_copy(src_hbm.at[pl.ds(0, _K)], srcv, si[s]).wait()
        pltpu.make_async_copy(src_hbm.at[pl.ds(0, _K)], dstv, si[s]).wait()

    def prep(s):
        srcv, dstv, snv, sdv = slots[s][0:4]
        for off in (0, 16, 24):
            sl = pl.ds(off, 16)
            dj = dstv[sl]
            mine = (dj >= lo) & (dj < lo + _NHALF)
            base = jnp.where(mine, dj - lo, _TRASH)
            snv[sl] = base
            sdv[sl] = jnp.where(mine, base + _DEN_OFF, _TRASH)

    def fire_gather(eb, s):
        srcv, dstv, _, _, dhv, ehv, bhv, cev = slots[s]
        pltpu.async_copy(dh_hbm.at[srcv], dhv, sg[s])
        pltpu.async_copy(eh_hbm.at[dstv], ehv, sg[s])
        pltpu.async_copy(bh_hbm.at[srcv], bhv, sg[s])
        pltpu.async_copy(ce_hbm.at[pl.ds(eb, _K)], cev, sg[s])

    def wait_gather(s):
        dhv = slots[s][4]
        for _ in range(4):
            pltpu.make_async_copy(ce_hbm.at[pl.ds(0, _K)], dhv, sg[s]).wait()

    def compute(s, carry):
        _, _, _, _, dhv, ehv, bhv, cev = slots[s]

        @plsc.parallel_loop(0, _K, 1, unroll=4, carry=tuple(carry))
        def row(i, rc):
            out = list(rc)
            for j in range(8):
                sl = pl.ds(j * 16, 16)
                ep = dhv[i, sl] + ehv[i, sl] + cev[i, sl]
                sgm = 1.0 / (1.0 + jnp.exp(-ep))
                bhv[i, sl] = bhv[i, sl] * sgm
                ehv[i, sl] = sgm
                if want_epre:
                    cev[i, sl] = ep
                    out[j] = rc[j] + ep
                    out[8 + j] = rc[8 + j] + ep * ep
            return tuple(out)
        return row

    def fire_scatter(eb, s):
        _, _, snv, sdv, dhv, ehv, bhv, cev = slots[s]
        pltpu.async_copy(bhv, acc.at[snv], sw[s], add=True)
        pltpu.async_copy(ehv, acc.at[sdv], sw[s], add=True)
        if want_epre:
            @pl.when(cid == s)
            def _():
                pltpu.async_copy(cev, ep_hbm.at[pl.ds(eb, _K)], sep[s])

    def wait_scatter(s):
        dhv = slots[s][4]
        for _ in range(2):
            pltpu.make_async_copy(ce_hbm.at[pl.ds(0, _K)], dhv, sw[s]).wait()
        if want_epre:
            @pl.when(cid == s)
            def _():
                pltpu.make_async_copy(ce_hbm.at[pl.ds(0, _K)], dhv, sep[s]).wait()

    fire_idx(base0, 0)
    fire_idx(base0 + _K, 1)
    wait_idx(0)
    prep(0)
    fire_gather(base0, 0)
    wait_idx(1)
    prep(1)
    fire_gather(base0 + _K, 1)

    def body(t, carry):
        for s in (0, 1):
            eb = base0 + (2 * t + s) * _K
            wait_gather(s)

            @pl.when(t < _NB2 - 1)
            def _():
                fire_idx(eb + 2 * _K, s)
            carry = compute(s, carry)
            fire_scatter(eb, s)
            wait_scatter(s)

            @pl.when(t < _NB2 - 1)
            def _():
                wait_idx(s)
                prep(s)
                fire_gather(eb + 2 * _K, s)
        return carry

    zero16 = jnp.zeros((16,), jnp.float32)
    carry = lax.fori_loop(0, _NB2, body, (zero16,) * 16)

    if want_epre:
        for j in range(8):
            sl = pl.ds(j * 16, 16)
            statv[0, sl] = carry[j]
            statv[1, sl] = carry[8 + j]
        pltpu.sync_copy(statv, st_hbm.at[pl.ds((cid * _NTILE + sid) * 8, 8)])

    plsc.subcore_barrier()
    pltpu.sync_copy(acc.at[pl.ds(sid * _TROWS, _TROWS)],
                    nd_hbm.at[pl.ds(cid * _ACC + sid * _TROWS, _TROWS)])


def _sc_edge(Bh, Dh, Eh, ce, src, dst, want_epre):
    mesh = plsc.VectorSubcoreMesh(core_axis_name="c", subcore_axis_name="s")
    out_type = [jax.ShapeDtypeStruct((2 * _ACC, H), jnp.float32)]
    if want_epre:
        out_type += [jax.ShapeDtypeStruct((E, H), jnp.float32),
                     jax.ShapeDtypeStruct((2 * _NTILE * 8, H), jnp.float32)]
    scratch = []
    for _s in range(2):
        scratch += [
            pltpu.VMEM((_K,), jnp.int32),
            pltpu.VMEM((_K,), jnp.int32),
            pltpu.VMEM((_K,), jnp.int32),
            pltpu.VMEM((_K,), jnp.int32),
            pltpu.VMEM((_K, H), jnp.float32),
            pltpu.VMEM((_K, H), jnp.float32),
            pltpu.VMEM((_K, H), jnp.float32),
            pltpu.VMEM((_K, H), jnp.float32),
        ]
    scratch += [
        pltpu.VMEM((8, H), jnp.float32),
        pltpu.VMEM_SHARED((_ACC, H), jnp.float32),
    ]
    scratch += [pltpu.SemaphoreType.DMA] * 8
    outs = pl.kernel(
        functools.partial(_sc_edge_body, want_epre),
        out_type=out_type, mesh=mesh, scratch_types=scratch,
    )(Bh, Dh, Eh, ce, src, dst)
    if want_epre:
        nd, ep, st = outs
        st = st.reshape(2, _NTILE, 8, H)
        sums = st[:, :, 0].sum((0, 1))
        sqs = st[:, :, 1].sum((0, 1))
        mu_v = sums / (2 * E)
        mu = mu_v.reshape(1, H)
        rstd = lax.rsqrt(jnp.maximum(sqs / (2 * E) - mu_v ** 2, 0.0) + 1e-5).reshape(1, H)
    else:
        (nd,) = outs
        ep = mu = rstd = None
    nd = nd.reshape(2, _ACC, H)
    num = jnp.concatenate([nd[0, :_NHALF], nd[1, :_NHALF]], axis=0)
    den = jnp.concatenate([nd[0, _DEN_OFF:_DEN_OFF + _NHALF],
                           nd[1, _DEN_OFF:_DEN_OFF + _NHALF]], axis=0)
    return num, den, ep, mu, rstd


def _sc_pq_body(p_hbm, q_hbm, src_hbm, dst_hbm, xp_hbm,
                srcv, dstv, pv, qv, xv, s0, s1, s2, s3):
    cid = lax.axis_index("c")
    sid = lax.axis_index("s")
    base0 = (cid * _NTILE + sid) * _EPT_PQ

    def chunk(ic, _):
        eb = base0 + ic * _KPQ
        ca = pltpu.async_copy(src_hbm.at[pl.ds(eb, _KPQ)], srcv, s0)
        cb = pltpu.async_copy(dst_hbm.at[pl.ds(eb, _KPQ)], dstv, s1)
        ca.wait()
        cb.wait()
        c1 = pltpu.async_copy(p_hbm.at[srcv], pv, s2)
        c2 = pltpu.async_copy(q_hbm.at[dstv], qv, s3)
        c1.wait()
        c2.wait()

        def row(i, _):
            for j in range(8):
                sl = pl.ds(j * 16, 16)
                xv[i, sl] = pv[i, sl] + qv[i, sl]
            return 0
        lax.fori_loop(0, _KPQ, row, 0)
        pltpu.sync_copy(xv, xp_hbm.at[pl.ds(eb, _KPQ)])
        return 0

    lax.fori_loop(0, _EPT_PQ // _KPQ, chunk, 0)


def _sc_pq(P, Q, src, dst):
    mesh = plsc.VectorSubcoreMesh(core_axis_name="c", subcore_axis_name="s")
    scratch = [
        pltpu.VMEM((_KPQ,), jnp.int32),
        pltpu.VMEM((_KPQ,), jnp.int32),
        pltpu.VMEM((_KPQ, H), jnp.float32),
        pltpu.VMEM((_KPQ, H), jnp.float32),
        pltpu.VMEM((_KPQ, H), jnp.float32),
        pltpu.SemaphoreType.DMA,
        pltpu.SemaphoreType.DMA,
        pltpu.SemaphoreType.DMA,
        pltpu.SemaphoreType.DMA,
    ]
    return pl.kernel(
        _sc_pq_body,
        out_type=jax.ShapeDtypeStruct((E, H), jnp.float32),
        mesh=mesh, scratch_types=scratch,
    )(P, Q, src, dst)


def kernel(h, e, edge_index, W_emb_h, b_emb_h, W_emb_e, b_emb_e, W_layers, b_layers,
           W_assign, b_assign, W_mlp0, b_mlp0, W_mlp1, b_mlp1, W_mlp2, b_mlp2):
    src = edge_index[0]
    dst = edge_index[1]

    hcur = _embed_h(h, W_emb_h, b_emb_h)
    ecur, ce = _edge_embed(e, W_emb_e, b_emb_e, W_layers[0, 2], b_layers[0, 2])

    s_list = []
    bi = 0
    for l in range(4):
        Wl, bl = W_layers[l], b_layers[l]
        Wstk = jnp.concatenate([Wl[0], Wl[1], Wl[3], Wl[4]], axis=1)
        bstk = jnp.concatenate([bl[0], bl[1], bl[3], bl[4]]).reshape(1, 4 * H)
        Ah, Bh, Dh, Eh = _node_mm(hcur, Wstk, bstk)
        num, den, ep, mu, rstd = _sc_edge(Bh, Dh, Eh, ce, src, dst,
                                          want_epre=(l < 3))
        hcur = _node_update(Ah, num, den, hcur)
        if l < 3:
            ecur, ce = _edge_update(ecur, ep, mu, rstd,
                                    W_layers[l + 1, 2], b_layers[l + 1, 2])
        if l in (1, 3):
            hcur, s = _bilinear(hcur, W_assign[bi], b_assign[bi])
            s_list.append(s)
            bi += 1

    S = jnp.stack(s_list, axis=0)
    P, Q = _readout_pq(hcur, W_mlp0)
    xp = _sc_pq(P, Q, src, dst)
    logits = _readout_mlp(xp, b_mlp0, W_mlp1, b_mlp1, W_mlp2, b_mlp2)
    return logits, S


# speedup vs baseline: 1.0672x; 1.0672x over previous
import functools

import jax
import jax.numpy as jnp
from jax import lax
from jax.experimental import pallas as pl
from jax.experimental.pallas import tpu as pltpu
from jax.experimental.pallas import tpu_sc as plsc

N = 10000
E = 320000
H = 128
A = 100
B_E = 8000

_NTILE = 16
_EPT = E // _NTILE
_K = 40
_NCHUNK = _EPT // _K
_NB2 = _NCHUNK // 2
_KPQ = 40
_NHALF = N // 2
_DEN_OFF = 5120
_TRASH = 10120
_ACC = 10240
_TROWS = _ACC // _NTILE
_EPT_PQ = E // 32


def _embed_h_body(x_ref, w_ref, b_ref, o_ref):
    o_ref[...] = jnp.dot(x_ref[...], w_ref[...], preferred_element_type=jnp.float32) + b_ref[...]


def _node_mm_body(x_ref, w_ref, b_ref, ah_ref, bh_ref, dh_ref, eh_ref):
    y = jnp.dot(x_ref[...], w_ref[...], preferred_element_type=jnp.float32) + b_ref[...]
    ah_ref[...] = y[:, 0:H]
    bh_ref[...] = y[:, H:2 * H]
    dh_ref[...] = y[:, 2 * H:3 * H]
    eh_ref[...] = y[:, 3 * H:4 * H]


def _node_update_body(ah_ref, num_ref, den_ref, hres_ref, o_ref):
    hn = ah_ref[...] + num_ref[...] / (den_ref[...] + 1e-6)
    mu = jnp.mean(hn, axis=0, keepdims=True)
    var = jnp.mean((hn - mu) ** 2, axis=0, keepdims=True)
    o_ref[...] = hres_ref[...] + jax.nn.relu((hn - mu) * lax.rsqrt(var + 1e-5))


def _edge_embed_body(e_ref, we_ref, be_ref, w2_ref, b2_ref, e0_ref, ce_ref):
    e0 = jnp.dot(e_ref[...], we_ref[...], preferred_element_type=jnp.float32) + be_ref[...]
    e0_ref[...] = e0
    ce_ref[...] = jnp.dot(e0, w2_ref[...], preferred_element_type=jnp.float32) + b2_ref[...]


def _edge_update_body(eres_ref, ep_ref, mu_ref, rstd_ref, w2_ref, b2_ref,
                      eo_ref, ce_ref):
    eo = eres_ref[...] + jax.nn.relu((ep_ref[...] - mu_ref[...]) * rstd_ref[...])
    eo_ref[...] = eo
    ce_ref[...] = jnp.dot(eo, w2_ref[...], preferred_element_type=jnp.float32) + b2_ref[...]


def _bilin_s_body(x_ref, wa_ref, ba_ref, s_ref):
    z = jnp.dot(x_ref[...], wa_ref[...], preferred_element_type=jnp.float32) + ba_ref[...]
    m = jnp.max(z, axis=-1, keepdims=True)
    ez = jnp.exp(z - m)
    s_ref[...] = ez / jnp.sum(ez, axis=-1, keepdims=True)


def _bilin_update_body(h_ref, s_ref, st_ref, o_ref):
    m = jnp.dot(st_ref[...], h_ref[...], preferred_element_type=jnp.float32)
    o_ref[...] = h_ref[...] + jnp.dot(s_ref[...], m, preferred_element_type=jnp.float32)


def _readout_pq_body(h_ref, w0_ref, p_ref, q_ref):
    p_ref[...] = jnp.dot(h_ref[...], w0_ref[0:H], preferred_element_type=jnp.float32)
    q_ref[...] = jnp.dot(h_ref[...], w0_ref[H:2 * H], preferred_element_type=jnp.float32)


def _mlp_body(xp_ref, b0_ref, w1_ref, b1_ref, w2_ref, b2_ref, o_ref):
    x = jax.nn.relu(xp_ref[...] + b0_ref[...])
    y = jax.nn.relu(jnp.dot(x, w1_ref[...], preferred_element_type=jnp.float32) + b1_ref[...])
    o_ref[...] = jnp.dot(y, w2_ref[...], preferred_element_type=jnp.float32) + b2_ref[...]


def _embed_h(h, W, b):
    return pl.pallas_call(
        _embed_h_body,
        out_shape=jax.ShapeDtypeStruct((N, H), jnp.float32),
    )(h, W, b.reshape(1, H))


def _node_mm(x, Wstk, bstk):
    return pl.pallas_call(
        _node_mm_body,
        out_shape=[jax.ShapeDtypeStruct((N, H), jnp.float32)] * 4,
    )(x, Wstk, bstk)


def _node_update(Ah, num, den, hres):
    return pl.pallas_call(
        _node_update_body,
        out_shape=jax.ShapeDtypeStruct((N, H), jnp.float32),
    )(Ah, num, den, hres)


def _edge_embed(e, Wemb, bemb, W2, b2):
    g = E // B_E
    return pl.pallas_call(
        _edge_embed_body,
        grid=(g,),
        in_specs=[
            pl.BlockSpec((B_E, 16), lambda i: (i, 0)),
            pl.BlockSpec((16, H), lambda i: (0, 0)),
            pl.BlockSpec((1, H), lambda i: (0, 0)),
            pl.BlockSpec((H, H), lambda i: (0, 0)),
            pl.BlockSpec((1, H), lambda i: (0, 0)),
        ],
        out_specs=[
            pl.BlockSpec((B_E, H), lambda i: (i, 0)),
            pl.BlockSpec((B_E, H), lambda i: (i, 0)),
        ],
        out_shape=[
            jax.ShapeDtypeStruct((E, H), jnp.float32),
            jax.ShapeDtypeStruct((E, H), jnp.float32),
        ],
    )(e, Wemb, bemb.reshape(1, H), W2, b2.reshape(1, H))


def _edge_update(eres, ep, mu, rstd, W2n, b2n):
    g = E // B_E
    return pl.pallas_call(
        _edge_update_body,
        grid=(g,),
        in_specs=[
            pl.BlockSpec((B_E, H), lambda i: (i, 0)),
            pl.BlockSpec((B_E, H), lambda i: (i, 0)),
            pl.BlockSpec((1, H), lambda i: (0, 0)),
            pl.BlockSpec((1, H), lambda i: (0, 0)),
            pl.BlockSpec((H, H), lambda i: (0, 0)),
            pl.BlockSpec((1, H), lambda i: (0, 0)),
        ],
        out_specs=[
            pl.BlockSpec((B_E, H), lambda i: (i, 0)),
            pl.BlockSpec((B_E, H), lambda i: (i, 0)),
        ],
        out_shape=[
            jax.ShapeDtypeStruct((E, H), jnp.float32),
            jax.ShapeDtypeStruct((E, H), jnp.float32),
        ],
    )(eres, ep, mu, rstd, W2n, b2n.reshape(1, H))


def _bilinear(h, Wa, ba):
    s = pl.pallas_call(
        _bilin_s_body,
        out_shape=jax.ShapeDtypeStruct((N, A), jnp.float32),
    )(h, Wa, ba.reshape(1, A))
    h_out = pl.pallas_call(
        _bilin_update_body,
        out_shape=jax.ShapeDtypeStruct((N, H), jnp.float32),
    )(h, s, s.T)
    return h_out, s


def _readout_pq(h, W0):
    return pl.pallas_call(
        _readout_pq_body,
        out_shape=[jax.ShapeDtypeStruct((N, H), jnp.float32)] * 2,
    )(h, W0)


def _readout_mlp(xp, b0, W1, b1, W2, b2):
    g = E // B_E
    return pl.pallas_call(
        _mlp_body,
        grid=(g,),
        in_specs=[
            pl.BlockSpec((B_E, H), lambda i: (i, 0)),
            pl.BlockSpec((1, H), lambda i: (0, 0)),
            pl.BlockSpec((H, H // 2), lambda i: (0, 0)),
            pl.BlockSpec((1, H // 2), lambda i: (0, 0)),
            pl.BlockSpec((H // 2, 2), lambda i: (0, 0)),
            pl.BlockSpec((1, 2), lambda i: (0, 0)),
        ],
        out_specs=pl.BlockSpec((B_E, 2), lambda i: (i, 0)),
        out_shape=jax.ShapeDtypeStruct((E, 2), jnp.float32),
    )(xp, b0.reshape(1, H), W1, b1.reshape(1, H // 2), W2, b2.reshape(1, 2))


def _sc_edge_body(want_epre, bh_hbm, dh_hbm, eh_hbm, ce_hbm, src_hbm, dst_hbm,
                  *refs):
    if want_epre:
        nd_hbm, ep_hbm, st_hbm = refs[:3]
        r = refs[3:]
    else:
        nd_hbm = refs[0]
        ep_hbm = st_hbm = None
        r = refs[1:]
    slots = (r[0:8], r[8:16])
    statv, acc = r[16], r[17]
    si = (r[18], r[19])
    sg = (r[20], r[21])
    sw = (r[22], r[23])
    sep = (r[24], r[25])
    cid = lax.axis_index("c")
    sid = lax.axis_index("s")

    dhv0 = slots[0][4]

    def zrow(i, _):
        for j in range(8):
            dhv0[i, pl.ds(j * 16, 16)] = jnp.zeros((16,), jnp.float32)
        return 0
    lax.fori_loop(0, _K, zrow, 0)
    for rr in range(_TROWS // _K):
        pltpu.sync_copy(dhv0, acc.at[pl.ds(sid * _TROWS + rr * _K, _K)])
    plsc.subcore_barrier()

    base0 = sid * _EPT
    lo = cid * _NHALF

    def fire_idx(eb, s):
        srcv, dstv = slots[s][0], slots[s][1]
        pltpu.async_copy(src_hbm.at[pl.ds(eb, _K)], srcv, si[s])
        pltpu.async_copy(dst_hbm.at[pl.ds(eb, _K)], dstv, si[s])

    def wait_idx(s):
        srcv, dstv = slots[s][0], slots[s][1]
        pltpu.make_async_copy(src_hbm.at[pl.ds(0, _K)], srcv, si[s]).wait()
        pltpu.make_async_copy(src_hbm.at[pl.ds(0, _K)], dstv, si[s]).wait()

    def prep(s):
        srcv, dstv, snv, sdv = slots[s][0:4]
        for off in (0, 16, 24):
            sl = pl.ds(off, 16)
            dj = dstv[sl]
            mine = (dj >= lo) & (dj < lo + _NHALF)
            base = jnp.where(mine, dj - lo, _TRASH)
            snv[sl] = base
            sdv[sl] = jnp.where(mine, base + _DEN_OFF, _TRASH)

    def fire_gather(eb, s):
        srcv, dstv, _, _, dhv, ehv, bhv, cev = slots[s]
        pltpu.async_copy(dh_hbm.at[srcv], dhv, sg[s])
        pltpu.async_copy(eh_hbm.at[dstv], ehv, sg[s])
        pltpu.async_copy(bh_hbm.at[srcv], bhv, sg[s])
        pltpu.async_copy(ce_hbm.at[pl.ds(eb, _K)], cev, sg[s])

    def wait_gather(s):
        dhv = slots[s][4]
        for _ in range(4):
            pltpu.make_async_copy(ce_hbm.at[pl.ds(0, _K)], dhv, sg[s]).wait()

    def compute(s, carry):
        _, _, _, _, dhv, ehv, bhv, cev = slots[s]

        def row(i, rc):
            out = list(rc)
            for j in range(8):
                sl = pl.ds(j * 16, 16)
                ep = dhv[i, sl] + ehv[i, sl] + cev[i, sl]
                sgm = 1.0 / (1.0 + jnp.exp(-ep))
                bhv[i, sl] = bhv[i, sl] * sgm
                ehv[i, sl] = sgm
                if want_epre:
                    cev[i, sl] = ep
                    out[j] = rc[j] + ep
                    out[8 + j] = rc[8 + j] + ep * ep
            return tuple(out)
        return lax.fori_loop(0, _K, row, carry)

    def fire_scatter(eb, s):
        _, _, snv, sdv, dhv, ehv, bhv, cev = slots[s]
        pltpu.async_copy(bhv, acc.at[snv], sw[s], add=True)
        pltpu.async_copy(ehv, acc.at[sdv], sw[s], add=True)
        if want_epre:
            @pl.when(cid == s)
            def _():
                pltpu.async_copy(cev, ep_hbm.at[pl.ds(eb, _K)], sep[s])

    def wait_scatter(s):
        dhv = slots[s][4]
        for _ in range(2):
            pltpu.make_async_copy(ce_hbm.at[pl.ds(0, _K)], dhv, sw[s]).wait()
        if want_epre:
            @pl.when(cid == s)
            def _():
                pltpu.make_async_copy(ce_hbm.at[pl.ds(0, _K)], dhv, sep[s]).wait()

    fire_idx(base0, 0)
    fire_idx(base0 + _K, 1)
    wait_idx(0)
    prep(0)
    fire_gather(base0, 0)
    wait_idx(1)
    prep(1)
    fire_gather(base0 + _K, 1)

    def body(t, carry):
        for s in (0, 1):
            eb = base0 + (2 * t + s) * _K
            wait_gather(s)

            @pl.when(t < _NB2 - 1)
            def _():
                fire_idx(eb + 2 * _K, s)
            carry = compute(s, carry)
            fire_scatter(eb, s)
            wait_scatter(s)

            @pl.when(t < _NB2 - 1)
            def _():
                wait_idx(s)
                prep(s)
                fire_gather(eb + 2 * _K, s)
        return carry

    zero16 = jnp.zeros((16,), jnp.float32)
    carry = lax.fori_loop(0, _NB2, body, (zero16,) * 16)

    if want_epre:
        for j in range(8):
            sl = pl.ds(j * 16, 16)
            statv[0, sl] = carry[j]
            statv[1, sl] = carry[8 + j]
        pltpu.sync_copy(statv, st_hbm.at[pl.ds((cid * _NTILE + sid) * 8, 8)])

    plsc.subcore_barrier()
    pltpu.sync_copy(acc.at[pl.ds(sid * _TROWS, _TROWS)],
                    nd_hbm.at[pl.ds(cid * _ACC + sid * _TROWS, _TROWS)])


def _sc_edge(Bh, Dh, Eh, ce, src, dst, want_epre):
    mesh = plsc.VectorSubcoreMesh(core_axis_name="c", subcore_axis_name="s")
    out_type = [jax.ShapeDtypeStruct((2 * _ACC, H), jnp.float32)]
    if want_epre:
        out_type += [jax.ShapeDtypeStruct((E, H), jnp.float32),
                     jax.ShapeDtypeStruct((2 * _NTILE * 8, H), jnp.float32)]
    scratch = []
    for _s in range(2):
        scratch += [
            pltpu.VMEM((_K,), jnp.int32),
            pltpu.VMEM((_K,), jnp.int32),
            pltpu.VMEM((_K,), jnp.int32),
            pltpu.VMEM((_K,), jnp.int32),
            pltpu.VMEM((_K, H), jnp.float32),
            pltpu.VMEM((_K, H), jnp.float32),
            pltpu.VMEM((_K, H), jnp.float32),
            pltpu.VMEM((_K, H), jnp.float32),
        ]
    scratch += [
        pltpu.VMEM((8, H), jnp.float32),
        pltpu.VMEM_SHARED((_ACC, H), jnp.float32),
    ]
    scratch += [pltpu.SemaphoreType.DMA] * 8
    outs = pl.kernel(
        functools.partial(_sc_edge_body, want_epre),
        out_type=out_type, mesh=mesh, scratch_types=scratch,
    )(Bh, Dh, Eh, ce, src, dst)
    if want_epre:
        nd, ep, st = outs
        st = st.reshape(2, _NTILE, 8, H)
        sums = st[:, :, 0].sum((0, 1))
        sqs = st[:, :, 1].sum((0, 1))
        mu_v = sums / (2 * E)
        mu = mu_v.reshape(1, H)
        rstd = lax.rsqrt(jnp.maximum(sqs / (2 * E) - mu_v ** 2, 0.0) + 1e-5).reshape(1, H)
    else:
        (nd,) = outs
        ep = mu = rstd = None
    nd = nd.reshape(2, _ACC, H)
    num = jnp.concatenate([nd[0, :_NHALF], nd[1, :_NHALF]], axis=0)
    den = jnp.concatenate([nd[0, _DEN_OFF:_DEN_OFF + _NHALF],
                           nd[1, _DEN_OFF:_DEN_OFF + _NHALF]], axis=0)
    return num, den, ep, mu, rstd


def _sc_pq_body(p_hbm, q_hbm, src_hbm, dst_hbm, xp_hbm, *r):
    slots = (r[0:4], r[4:8])
    si = (r[8], r[9])
    sg = (r[10], r[11])
    sw = (r[12], r[13])
    cid = lax.axis_index("c")
    sid = lax.axis_index("s")
    base0 = (cid * _NTILE + sid) * _EPT_PQ
    nb2 = _EPT_PQ // _KPQ // 2

    def fire_idx(eb, s):
        srcv, dstv = slots[s][0], slots[s][1]
        pltpu.async_copy(src_hbm.at[pl.ds(eb, _KPQ)], srcv, si[s])
        pltpu.async_copy(dst_hbm.at[pl.ds(eb, _KPQ)], dstv, si[s])

    def wait_idx(s):
        srcv = slots[s][0]
        for _ in range(2):
            pltpu.make_async_copy(src_hbm.at[pl.ds(0, _KPQ)], srcv, si[s]).wait()

    def fire_gather(s):
        srcv, dstv, pv, qv = slots[s]
        pltpu.async_copy(p_hbm.at[srcv], pv, sg[s])
        pltpu.async_copy(q_hbm.at[dstv], qv, sg[s])

    def wait_gather(s):
        pv = slots[s][2]
        for _ in range(2):
            pltpu.make_async_copy(p_hbm.at[pl.ds(0, _KPQ)], pv, sg[s]).wait()

    def compute(s):
        _, _, pv, qv = slots[s]

        def row(i, _):
            for j in range(8):
                sl = pl.ds(j * 16, 16)
                pv[i, sl] = pv[i, sl] + qv[i, sl]
            return 0
        lax.fori_loop(0, _KPQ, row, 0)

    def wait_store(s):
        pv = slots[s][2]
        pltpu.make_async_copy(p_hbm.at[pl.ds(0, _KPQ)], pv, sw[s]).wait()

    fire_idx(base0, 0)
    fire_idx(base0 + _KPQ, 1)
    wait_idx(0)
    fire_gather(0)
    wait_idx(1)
    fire_gather(1)

    def body(t, _):
        for s in (0, 1):
            eb = base0 + (2 * t + s) * _KPQ
            wait_gather(s)

            @pl.when(t < nb2 - 1)
            def _():
                fire_idx(eb + 2 * _KPQ, s)
            compute(s)
            pltpu.async_copy(slots[s][2], xp_hbm.at[pl.ds(eb, _KPQ)], sw[s])
            wait_store(s)

            @pl.when(t < nb2 - 1)
            def _():
                wait_idx(s)
                fire_gather(s)
        return 0

    lax.fori_loop(0, nb2, body, 0)


def _sc_pq(P, Q, src, dst):
    mesh = plsc.VectorSubcoreMesh(core_axis_name="c", subcore_axis_name="s")
    scratch = []
    for _s in range(2):
        scratch += [
            pltpu.VMEM((_KPQ,), jnp.int32),
            pltpu.VMEM((_KPQ,), jnp.int32),
            pltpu.VMEM((_KPQ, H), jnp.float32),
            pltpu.VMEM((_KPQ, H), jnp.float32),
        ]
    scratch += [pltpu.SemaphoreType.DMA] * 6
    return pl.kernel(
        _sc_pq_body,
        out_type=jax.ShapeDtypeStruct((E, H), jnp.float32),
        mesh=mesh, scratch_types=scratch,
    )(P, Q, src, dst)


def kernel(h, e, edge_index, W_emb_h, b_emb_h, W_emb_e, b_emb_e, W_layers, b_layers,
           W_assign, b_assign, W_mlp0, b_mlp0, W_mlp1, b_mlp1, W_mlp2, b_mlp2):
    src = edge_index[0]
    dst = edge_index[1]

    hcur = _embed_h(h, W_emb_h, b_emb_h)
    ecur, ce = _edge_embed(e, W_emb_e, b_emb_e, W_layers[0, 2], b_layers[0, 2])

    s_list = []
    bi = 0
    for l in range(4):
        Wl, bl = W_layers[l], b_layers[l]
        Wstk = jnp.concatenate([Wl[0], Wl[1], Wl[3], Wl[4]], axis=1)
        bstk = jnp.concatenate([bl[0], bl[1], bl[3], bl[4]]).reshape(1, 4 * H)
        Ah, Bh, Dh, Eh = _node_mm(hcur, Wstk, bstk)
        num, den, ep, mu, rstd = _sc_edge(Bh, Dh, Eh, ce, src, dst,
                                          want_epre=(l < 3))
        hcur = _node_update(Ah, num, den, hcur)
        if l < 3:
            ecur, ce = _edge_update(ecur, ep, mu, rstd,
                                    W_layers[l + 1, 2], b_layers[l + 1, 2])
        if l in (1, 3):
            hcur, s = _bilinear(hcur, W_assign[bi], b_assign[bi])
            s_list.append(s)
            bi += 1

    S = jnp.stack(s_list, axis=0)
    P, Q = _readout_pq(hcur, W_mlp0)
    xp = _sc_pq(P, Q, src, dst)
    logits = _readout_mlp(xp, b_mlp0, W_mlp1, b_mlp1, W_mlp2, b_mlp2)
    return logits, S


# speedup vs baseline: 1.0732x; 1.0056x over previous
import functools

import jax
import jax.numpy as jnp
from jax import lax
from jax.experimental import pallas as pl
from jax.experimental.pallas import tpu as pltpu
from jax.experimental.pallas import tpu_sc as plsc

N = 10000
E = 320000
H = 128
A = 100
B_E = 8000

_NTILE = 16
_EPT = E // _NTILE
_K = 40
_NCHUNK = _EPT // _K
_NB2 = _NCHUNK // 2
_KPQ = 40
_NHALF = N // 2
_DEN_OFF = 5120
_TRASH = 10120
_ACC = 10240
_TROWS = _ACC // _NTILE
_EPT_PQ = E // 32


def _embed_h_body(x_ref, w_ref, b_ref, o_ref):
    o_ref[...] = jnp.dot(x_ref[...], w_ref[...], preferred_element_type=jnp.float32) + b_ref[...]


def _node_mm_body(x_ref, w_ref, b_ref, ah_ref, bh_ref, dh_ref, eh_ref):
    y = jnp.dot(x_ref[...], w_ref[...], preferred_element_type=jnp.float32) + b_ref[...]
    ah_ref[...] = y[:, 0:H]
    bh_ref[...] = y[:, H:2 * H]
    dh_ref[...] = y[:, 2 * H:3 * H]
    eh_ref[...] = y[:, 3 * H:4 * H]


def _node_update_body(ah_ref, num_ref, den_ref, hres_ref, o_ref):
    hn = ah_ref[...] + num_ref[...] / (den_ref[...] + 1e-6)
    mu = jnp.mean(hn, axis=0, keepdims=True)
    var = jnp.mean((hn - mu) ** 2, axis=0, keepdims=True)
    o_ref[...] = hres_ref[...] + jax.nn.relu((hn - mu) * lax.rsqrt(var + 1e-5))


def _node_update_mm_body(ah_ref, num_ref, den_ref, hres_ref, w_ref, b_ref,
                         o_ref, ah2_ref, bh2_ref, dh2_ref, eh2_ref):
    hn = ah_ref[...] + num_ref[...] / (den_ref[...] + 1e-6)
    mu = jnp.mean(hn, axis=0, keepdims=True)
    var = jnp.mean((hn - mu) ** 2, axis=0, keepdims=True)
    ho = hres_ref[...] + jax.nn.relu((hn - mu) * lax.rsqrt(var + 1e-5))
    o_ref[...] = ho
    y = jnp.dot(ho, w_ref[...], preferred_element_type=jnp.float32) + b_ref[...]
    ah2_ref[...] = y[:, 0:H]
    bh2_ref[...] = y[:, H:2 * H]
    dh2_ref[...] = y[:, 2 * H:3 * H]
    eh2_ref[...] = y[:, 3 * H:4 * H]


def _edge_embed_body(e_ref, we_ref, be_ref, w2_ref, b2_ref, e0_ref, ce_ref):
    e0 = jnp.dot(e_ref[...], we_ref[...], preferred_element_type=jnp.float32) + be_ref[...]
    e0_ref[...] = e0
    ce_ref[...] = jnp.dot(e0, w2_ref[...], preferred_element_type=jnp.float32) + b2_ref[...]


def _edge_update_body(eres_ref, ep_ref, mu_ref, rstd_ref, w2_ref, b2_ref,
                      eo_ref, ce_ref):
    eo = eres_ref[...] + jax.nn.relu((ep_ref[...] - mu_ref[...]) * rstd_ref[...])
    eo_ref[...] = eo
    ce_ref[...] = jnp.dot(eo, w2_ref[...], preferred_element_type=jnp.float32) + b2_ref[...]


def _bilin_s_body(x_ref, wa_ref, ba_ref, s_ref):
    z = jnp.dot(x_ref[...], wa_ref[...], preferred_element_type=jnp.float32) + ba_ref[...]
    m = jnp.max(z, axis=-1, keepdims=True)
    ez = jnp.exp(z - m)
    s_ref[...] = ez / jnp.sum(ez, axis=-1, keepdims=True)


def _bilin_update_body(h_ref, s_ref, st_ref, o_ref):
    m = jnp.dot(st_ref[...], h_ref[...], preferred_element_type=jnp.float32)
    o_ref[...] = h_ref[...] + jnp.dot(s_ref[...], m, preferred_element_type=jnp.float32)


def _readout_pq_body(h_ref, w0_ref, p_ref, q_ref):
    p_ref[...] = jnp.dot(h_ref[...], w0_ref[0:H], preferred_element_type=jnp.float32)
    q_ref[...] = jnp.dot(h_ref[...], w0_ref[H:2 * H], preferred_element_type=jnp.float32)


def _mlp_body(xp_ref, b0_ref, w1_ref, b1_ref, w2_ref, b2_ref, o_ref):
    x = jax.nn.relu(xp_ref[...] + b0_ref[...])
    y = jax.nn.relu(jnp.dot(x, w1_ref[...], preferred_element_type=jnp.float32) + b1_ref[...])
    o_ref[...] = jnp.dot(y, w2_ref[...], preferred_element_type=jnp.float32) + b2_ref[...]


def _embed_h(h, W, b):
    return pl.pallas_call(
        _embed_h_body,
        out_shape=jax.ShapeDtypeStruct((N, H), jnp.float32),
    )(h, W, b.reshape(1, H))


def _node_mm(x, Wstk, bstk):
    return pl.pallas_call(
        _node_mm_body,
        out_shape=[jax.ShapeDtypeStruct((N, H), jnp.float32)] * 4,
    )(x, Wstk, bstk)


def _node_update(Ah, num, den, hres):
    return pl.pallas_call(
        _node_update_body,
        out_shape=jax.ShapeDtypeStruct((N, H), jnp.float32),
    )(Ah, num, den, hres)


def _node_update_mm(Ah, num, den, hres, Wstk, bstk):
    return pl.pallas_call(
        _node_update_mm_body,
        out_shape=[jax.ShapeDtypeStruct((N, H), jnp.float32)] * 5,
    )(Ah, num, den, hres, Wstk, bstk)


def _edge_embed(e, Wemb, bemb, W2, b2):
    g = E // B_E
    return pl.pallas_call(
        _edge_embed_body,
        grid=(g,),
        in_specs=[
            pl.BlockSpec((B_E, 16), lambda i: (i, 0)),
            pl.BlockSpec((16, H), lambda i: (0, 0)),
            pl.BlockSpec((1, H), lambda i: (0, 0)),
            pl.BlockSpec((H, H), lambda i: (0, 0)),
            pl.BlockSpec((1, H), lambda i: (0, 0)),
        ],
        out_specs=[
            pl.BlockSpec((B_E, H), lambda i: (i, 0)),
            pl.BlockSpec((B_E, H), lambda i: (i, 0)),
        ],
        out_shape=[
            jax.ShapeDtypeStruct((E, H), jnp.float32),
            jax.ShapeDtypeStruct((E, H), jnp.float32),
        ],
    )(e, Wemb, bemb.reshape(1, H), W2, b2.reshape(1, H))


def _edge_update(eres, ep, mu, rstd, W2n, b2n):
    g = E // B_E
    return pl.pallas_call(
        _edge_update_body,
        grid=(g,),
        in_specs=[
            pl.BlockSpec((B_E, H), lambda i: (i, 0)),
            pl.BlockSpec((B_E, H), lambda i: (i, 0)),
            pl.BlockSpec((1, H), lambda i: (0, 0)),
            pl.BlockSpec((1, H), lambda i: (0, 0)),
            pl.BlockSpec((H, H), lambda i: (0, 0)),
            pl.BlockSpec((1, H), lambda i: (0, 0)),
        ],
        out_specs=[
            pl.BlockSpec((B_E, H), lambda i: (i, 0)),
            pl.BlockSpec((B_E, H), lambda i: (i, 0)),
        ],
        out_shape=[
            jax.ShapeDtypeStruct((E, H), jnp.float32),
            jax.ShapeDtypeStruct((E, H), jnp.float32),
        ],
    )(eres, ep, mu, rstd, W2n, b2n.reshape(1, H))


def _bilinear(h, Wa, ba):
    s = pl.pallas_call(
        _bilin_s_body,
        out_shape=jax.ShapeDtypeStruct((N, A), jnp.float32),
    )(h, Wa, ba.reshape(1, A))
    h_out = pl.pallas_call(
        _bilin_update_body,
        out_shape=jax.ShapeDtypeStruct((N, H), jnp.float32),
    )(h, s, s.T)
    return h_out, s


def _readout_pq(h, W0):
    return pl.pallas_call(
        _readout_pq_body,
        out_shape=[jax.ShapeDtypeStruct((N, H), jnp.float32)] * 2,
    )(h, W0)


def _readout_mlp(xp, b0, W1, b1, W2, b2):
    g = E // B_E
    return pl.pallas_call(
        _mlp_body,
        grid=(g,),
        in_specs=[
            pl.BlockSpec((B_E, H), lambda i: (i, 0)),
            pl.BlockSpec((1, H), lambda i: (0, 0)),
            pl.BlockSpec((H, H // 2), lambda i: (0, 0)),
            pl.BlockSpec((1, H // 2), lambda i: (0, 0)),
            pl.BlockSpec((H // 2, 2), lambda i: (0, 0)),
            pl.BlockSpec((1, 2), lambda i: (0, 0)),
        ],
        out_specs=pl.BlockSpec((B_E, 2), lambda i: (i, 0)),
        out_shape=jax.ShapeDtypeStruct((E, 2), jnp.float32),
    )(xp, b0.reshape(1, H), W1, b1.reshape(1, H // 2), W2, b2.reshape(1, 2))


def _sc_edge_body(want_epre, bh_hbm, dh_hbm, eh_hbm, ce_hbm, src_hbm, dst_hbm,
                  *refs):
    if want_epre:
        nd_hbm, ep_hbm, st_hbm = refs[:3]
        r = refs[3:]
    else:
        nd_hbm = refs[0]
        ep_hbm = st_hbm = None
        r = refs[1:]
    slots = (r[0:8], r[8:16])
    statv, acc = r[16], r[17]
    si = (r[18], r[19])
    sg = (r[20], r[21])
    sw = (r[22], r[23])
    sep = (r[24], r[25])
    cid = lax.axis_index("c")
    sid = lax.axis_index("s")

    dhv0 = slots[0][4]

    def zrow(i, _):
        for j in range(8):
            dhv0[i, pl.ds(j * 16, 16)] = jnp.zeros((16,), jnp.float32)
        return 0
    lax.fori_loop(0, _K, zrow, 0)
    for rr in range(_TROWS // _K):
        pltpu.sync_copy(dhv0, acc.at[pl.ds(sid * _TROWS + rr * _K, _K)])
    plsc.subcore_barrier()

    base0 = sid * _EPT
    lo = cid * _NHALF

    def fire_idx(eb, s):
        srcv, dstv = slots[s][0], slots[s][1]
        pltpu.async_copy(src_hbm.at[pl.ds(eb, _K)], srcv, si[s])
        pltpu.async_copy(dst_hbm.at[pl.ds(eb, _K)], dstv, si[s])

    def wait_idx(s):
        srcv, dstv = slots[s][0], slots[s][1]
        pltpu.make_async_copy(src_hbm.at[pl.ds(0, _K)], srcv, si[s]).wait()
        pltpu.make_async_copy(src_hbm.at[pl.ds(0, _K)], dstv, si[s]).wait()

    def prep(s):
        srcv, dstv, snv, sdv = slots[s][0:4]
        for off in (0, 16, 24):
            sl = pl.ds(off, 16)
            dj = dstv[sl]
            mine = (dj >= lo) & (dj < lo + _NHALF)
            base = jnp.where(mine, dj - lo, _TRASH)
            snv[sl] = base
            sdv[sl] = jnp.where(mine, base + _DEN_OFF, _TRASH)

    def fire_gather(eb, s):
        srcv, dstv, _, _, dhv, ehv, bhv, cev = slots[s]
        pltpu.async_copy(dh_hbm.at[srcv], dhv, sg[s])
        pltpu.async_copy(eh_hbm.at[dstv], ehv, sg[s])
        pltpu.async_copy(bh_hbm.at[srcv], bhv, sg[s])
        pltpu.async_copy(ce_hbm.at[pl.ds(eb, _K)], cev, sg[s])

    def wait_gather(s):
        dhv = slots[s][4]
        for _ in range(4):
            pltpu.make_async_copy(ce_hbm.at[pl.ds(0, _K)], dhv, sg[s]).wait()

    def compute(s, carry):
        _, _, _, _, dhv, ehv, bhv, cev = slots[s]

        def row(i, rc):
            out = list(rc)
            for j in range(8):
                sl = pl.ds(j * 16, 16)
                ep = dhv[i, sl] + ehv[i, sl] + cev[i, sl]
                sgm = 1.0 / (1.0 + jnp.exp(-ep))
                bhv[i, sl] = bhv[i, sl] * sgm
                ehv[i, sl] = sgm
                if want_epre:
                    cev[i, sl] = ep
                    out[j] = rc[j] + ep
                    out[8 + j] = rc[8 + j] + ep * ep
            return tuple(out)
        return lax.fori_loop(0, _K, row, carry)

    def fire_scatter(eb, s):
        _, _, snv, sdv, dhv, ehv, bhv, cev = slots[s]
        pltpu.async_copy(bhv, acc.at[snv], sw[s], add=True)
        pltpu.async_copy(ehv, acc.at[sdv], sw[s], add=True)
        if want_epre:
            @pl.when(cid == s)
            def _():
                pltpu.async_copy(cev, ep_hbm.at[pl.ds(eb, _K)], sep[s])

    def wait_scatter(s):
        dhv = slots[s][4]
        for _ in range(2):
            pltpu.make_async_copy(ce_hbm.at[pl.ds(0, _K)], dhv, sw[s]).wait()
        if want_epre:
            @pl.when(cid == s)
            def _():
                pltpu.make_async_copy(ce_hbm.at[pl.ds(0, _K)], dhv, sep[s]).wait()

    fire_idx(base0, 0)
    fire_idx(base0 + _K, 1)
    wait_idx(0)
    prep(0)
    fire_gather(base0, 0)
    wait_idx(1)
    prep(1)
    fire_gather(base0 + _K, 1)

    def body(t, carry):
        for s in (0, 1):
            eb = base0 + (2 * t + s) * _K
            wait_gather(s)

            @pl.when(t < _NB2 - 1)
            def _():
                fire_idx(eb + 2 * _K, s)
            carry = compute(s, carry)
            fire_scatter(eb, s)
            wait_scatter(s)

            @pl.when(t < _NB2 - 1)
            def _():
                wait_idx(s)
                prep(s)
                fire_gather(eb + 2 * _K, s)
        return carry

    zero16 = jnp.zeros((16,), jnp.float32)
    carry = lax.fori_loop(0, _NB2, body, (zero16,) * 16)

    if want_epre:
        for j in range(8):
            sl = pl.ds(j * 16, 16)
            statv[0, sl] = carry[j]
            statv[1, sl] = carry[8 + j]
        pltpu.sync_copy(statv, st_hbm.at[pl.ds((cid * _NTILE + sid) * 8, 8)])

    plsc.subcore_barrier()
    pltpu.sync_copy(acc.at[pl.ds(sid * _TROWS, _TROWS)],
                    nd_hbm.at[pl.ds(cid * _ACC + sid * _TROWS, _TROWS)])


def _sc_edge(Bh, Dh, Eh, ce, src, dst, want_epre):
    mesh = plsc.VectorSubcoreMesh(core_axis_name="c", subcore_axis_name="s")
    out_type = [jax.ShapeDtypeStruct((2 * _ACC, H), jnp.float32)]
    if want_epre:
        out_type += [jax.ShapeDtypeStruct((E, H), jnp.float32),
                     jax.ShapeDtypeStruct((2 * _NTILE * 8, H), jnp.float32)]
    scratch = []
    for _s in range(2):
        scratch += [
            pltpu.VMEM((_K,), jnp.int32),
            pltpu.VMEM((_K,), jnp.int32),
            pltpu.VMEM((_K,), jnp.int32),
            pltpu.VMEM((_K,), jnp.int32),
            pltpu.VMEM((_K, H), jnp.float32),
            pltpu.VMEM((_K, H), jnp.float32),
            pltpu.VMEM((_K, H), jnp.float32),
            pltpu.VMEM((_K, H), jnp.float32),
        ]
    scratch += [
        pltpu.VMEM((8, H), jnp.float32),
        pltpu.VMEM_SHARED((_ACC, H), jnp.float32),
    ]
    scratch += [pltpu.SemaphoreType.DMA] * 8
    outs = pl.kernel(
        functools.partial(_sc_edge_body, want_epre),
        out_type=out_type, mesh=mesh, scratch_types=scratch,
    )(Bh, Dh, Eh, ce, src, dst)
    if want_epre:
        nd, ep, st = outs
        st = st.reshape(2, _NTILE, 8, H)
        sums = st[:, :, 0].sum((0, 1))
        sqs = st[:, :, 1].sum((0, 1))
        mu_v = sums / (2 * E)
        mu = mu_v.reshape(1, H)
        rstd = lax.rsqrt(jnp.maximum(sqs / (2 * E) - mu_v ** 2, 0.0) + 1e-5).reshape(1, H)
    else:
        (nd,) = outs
        ep = mu = rstd = None
    nd = nd.reshape(2, _ACC, H)
    num = jnp.concatenate([nd[0, :_NHALF], nd[1, :_NHALF]], axis=0)
    den = jnp.concatenate([nd[0, _DEN_OFF:_DEN_OFF + _NHALF],
                           nd[1, _DEN_OFF:_DEN_OFF + _NHALF]], axis=0)
    return num, den, ep, mu, rstd


def _sc_pq_body(p_hbm, q_hbm, src_hbm, dst_hbm, xp_hbm, *r):
    slots = (r[0:4], r[4:8])
    si = (r[8], r[9])
    sg = (r[10], r[11])
    sw = (r[12], r[13])
    cid = lax.axis_index("c")
    sid = lax.axis_index("s")
    base0 = (cid * _NTILE + sid) * _EPT_PQ
    nb2 = _EPT_PQ // _KPQ // 2

    def fire_idx(eb, s):
        srcv, dstv = slots[s][0], slots[s][1]
        pltpu.async_copy(src_hbm.at[pl.ds(eb, _KPQ)], srcv, si[s])
        pltpu.async_copy(dst_hbm.at[pl.ds(eb, _KPQ)], dstv, si[s])

    def wait_idx(s):
        srcv = slots[s][0]
        for _ in range(2):
            pltpu.make_async_copy(src_hbm.at[pl.ds(0, _KPQ)], srcv, si[s]).wait()

    def fire_gather(s):
        srcv, dstv, pv, qv = slots[s]
        pltpu.async_copy(p_hbm.at[srcv], pv, sg[s])
        pltpu.async_copy(q_hbm.at[dstv], qv, sg[s])

    def wait_gather(s):
        pv = slots[s][2]
        for _ in range(2):
            pltpu.make_async_copy(p_hbm.at[pl.ds(0, _KPQ)], pv, sg[s]).wait()

    def compute(s):
        _, _, pv, qv = slots[s]

        def row(i, _):
            for j in range(8):
                sl = pl.ds(j * 16, 16)
                pv[i, sl] = pv[i, sl] + qv[i, sl]
            return 0
        lax.fori_loop(0, _KPQ, row, 0)

    def wait_store(s):
        pv = slots[s][2]
        pltpu.make_async_copy(p_hbm.at[pl.ds(0, _KPQ)], pv, sw[s]).wait()

    fire_idx(base0, 0)
    fire_idx(base0 + _KPQ, 1)
    wait_idx(0)
    fire_gather(0)
    wait_idx(1)
    fire_gather(1)

    def body(t, _):
        for s in (0, 1):
            eb = base0 + (2 * t + s) * _KPQ
            wait_gather(s)

            @pl.when(t < nb2 - 1)
            def _():
                fire_idx(eb + 2 * _KPQ, s)
            compute(s)
            pltpu.async_copy(slots[s][2], xp_hbm.at[pl.ds(eb, _KPQ)], sw[s])
            wait_store(s)

            @pl.when(t < nb2 - 1)
            def _():
                wait_idx(s)
                fire_gather(s)
        return 0

    lax.fori_loop(0, nb2, body, 0)


def _sc_pq(P, Q, src, dst):
    mesh = plsc.VectorSubcoreMesh(core_axis_name="c", subcore_axis_name="s")
    scratch = []
    for _s in range(2):
        scratch += [
            pltpu.VMEM((_KPQ,), jnp.int32),
            pltpu.VMEM((_KPQ,), jnp.int32),
            pltpu.VMEM((_KPQ, H), jnp.float32),
            pltpu.VMEM((_KPQ, H), jnp.float32),
        ]
    scratch += [pltpu.SemaphoreType.DMA] * 6
    return pl.kernel(
        _sc_pq_body,
        out_type=jax.ShapeDtypeStruct((E, H), jnp.float32),
        mesh=mesh, scratch_types=scratch,
    )(P, Q, src, dst)


def kernel(h, e, edge_index, W_emb_h, b_emb_h, W_emb_e, b_emb_e, W_layers, b_layers,
           W_assign, b_assign, W_mlp0, b_mlp0, W_mlp1, b_mlp1, W_mlp2, b_mlp2):
    src = edge_index[0]
    dst = edge_index[1]

    hcur = _embed_h(h, W_emb_h, b_emb_h)
    ecur, ce = _edge_embed(e, W_emb_e, b_emb_e, W_layers[0, 2], b_layers[0, 2])

    s_list = []
    bi = 0
    Wstks, bstks = [], []
    for l in range(4):
        Wl, bl = W_layers[l], b_layers[l]
        Wstks.append(jnp.concatenate([Wl[0], Wl[1], Wl[3], Wl[4]], axis=1))
        bstks.append(jnp.concatenate([bl[0], bl[1], bl[3], bl[4]]).reshape(1, 4 * H))
    Ah, Bh, Dh, Eh = _node_mm(hcur, Wstks[0], bstks[0])
    for l in range(4):
        num, den, ep, mu, rstd = _sc_edge(Bh, Dh, Eh, ce, src, dst,
                                          want_epre=(l < 3))
        if l in (0, 2):
            hcur, Ah, Bh, Dh, Eh = _node_update_mm(Ah, num, den, hcur,
                                                   Wstks[l + 1], bstks[l + 1])
        else:
            hcur = _node_update(Ah, num, den, hcur)
            hcur, s = _bilinear(hcur, W_assign[bi], b_assign[bi])
            s_list.append(s)
            bi += 1
            if l == 1:
                Ah, Bh, Dh, Eh = _node_mm(hcur, Wstks[2], bstks[2])
        if l < 3:
            ecur, ce = _edge_update(ecur, ep, mu, rstd,
                                    W_layers[l + 1, 2], b_layers[l + 1, 2])

    S = jnp.stack(s_list, axis=0)
    P, Q = _readout_pq(hcur, W_mlp0)
    xp = _sc_pq(P, Q, src, dst)
    logits = _readout_mlp(xp, b_mlp0, W_mlp1, b_mlp1, W_mlp2, b_mlp2)
    return logits, S
